# Initial kernel scaffold; baseline (speedup 1.0000x reference)
import functools
import jax
import jax.numpy as jnp
from jax import lax
from jax.experimental import pallas as pl
from jax.experimental.pallas import tpu as pltpu
from jax.experimental.pallas import tpu_sc as plsc

EMB = 512
T_TOKENS = 32768
B_BAGS = 16
TYPES = 128
NC = 2
NS = 16
NW = NC * NS
TPW = T_TOKENS // NW
CHUNK = 64
NCHUNK = TPW // CHUNK


def _sc_body(emb_hbm, feat_hbm, off_hbm, zeros_hbm, out_hbm,
             idx_v, seg_v, rows_v, acc_v, off_v, sem):
    wid = lax.axis_index("s") * NC + lax.axis_index("c")
    base = wid * TPW

    pltpu.sync_copy(off_hbm, off_v)
    pltpu.sync_copy(zeros_hbm, acc_v)

    off_splat = [
        plsc.load_gather(off_v, [jnp.full((16,), b, jnp.int32)])
        for b in range(B_BAGS)
    ]

    def chunk_body(c, _):
        tbase = pl.multiple_of(base + c * CHUNK, CHUNK)
        pltpu.sync_copy(feat_hbm.at[pl.ds(tbase, CHUNK)], idx_v)
        for g in range(CHUNK // 16):
            tvec = tbase + g * 16 + lax.iota(jnp.int32, 16)
            seg = jnp.full((16,), -1, jnp.int32)
            for b in range(B_BAGS):
                seg = seg + jnp.where(tvec >= off_splat[b], 1, 0).astype(jnp.int32)
            seg_v[pl.ds(g * 16, 16)] = seg
        pltpu.async_copy(emb_hbm.at[idx_v], rows_v, sem).wait()
        pltpu.sync_copy(rows_v, acc_v.at[seg_v], add=True)
        return 0

    lax.fori_loop(0, NCHUNK, chunk_body, 0)
    pltpu.sync_copy(acc_v, out_hbm.at[wid])


def _make_sc_kernel():
    mesh = plsc.VectorSubcoreMesh(core_axis_name="c", subcore_axis_name="s")
    return pl.kernel(
        _sc_body,
        out_type=jax.ShapeDtypeStruct((NW, B_BAGS, EMB), jnp.float32),
        mesh=mesh,
        scratch_types=[
            pltpu.VMEM((CHUNK,), jnp.int32),
            pltpu.VMEM((CHUNK,), jnp.int32),
            pltpu.VMEM((CHUNK, EMB), jnp.float32),
            pltpu.VMEM((B_BAGS, EMB), jnp.float32),
            pltpu.VMEM((B_BAGS,), jnp.int32),
            pltpu.SemaphoreType.DMA,
        ],
    )


def _tc_body(part_ref, off_ref, lin_ref, out_ref):
    sums = jnp.sum(part_ref[...], axis=0)
    off = off_ref[...]
    nxt = jnp.concatenate(
        [off[:, 1:], jnp.full((1, 1), T_TOKENS, jnp.int32)], axis=1)
    counts = (nxt - off).astype(jnp.float32)
    mean = sums / jnp.maximum(counts, 1.0).reshape(B_BAGS, 1)
    out_ref[...] = lax.dot_general(
        mean, lin_ref[...], (((1,), (1,)), ((), ())),
        preferred_element_type=jnp.float32)


@jax.jit
def kernel(feature_seq, offset_seq, emb_weight, lin_weight):
    zeros = jnp.zeros((B_BAGS, EMB), jnp.float32)
    partials = _make_sc_kernel()(emb_weight, feature_seq, offset_seq, zeros)
    return pl.pallas_call(
        _tc_body,
        out_shape=jax.ShapeDtypeStruct((B_BAGS, TYPES), jnp.float32),
    )(partials, offset_seq.reshape(1, B_BAGS), lin_weight)


# speedup vs baseline: 3.8554x; 3.8554x over previous
import jax
import jax.numpy as jnp
from jax import lax
from jax.experimental import pallas as pl
from jax.experimental.pallas import tpu as pltpu
from jax.experimental.pallas import tpu_sc as plsc

EMB = 512
NV = EMB // 16
T_TOKENS = 32768
B_BAGS = 16
TYPES = 128
NC = 2
NS = 16
NW = NC * NS
TPW = T_TOKENS // NW
CHUNK = 64
NCHUNK = TPW // CHUNK


def _sc_body(emb_hbm, feat_hbm, offb_hbm, zeros_hbm, out_hbm,
             idx_v, rows_v, acc_v, offb_v, sem):
    cid = lax.axis_index("c")
    sid = lax.axis_index("s")
    wid = sid * NC + cid
    base = wid * TPW

    pltpu.sync_copy(offb_hbm, offb_v)
    pltpu.sync_copy(zeros_hbm, acc_v)

    off_s = [lax.reduce_max(offb_v[b, :], (0,)) for b in range(B_BAGS)]
    off_s.append(jnp.int32(T_TOKENS))

    def chunk_body(c, _):
        tbase = pl.multiple_of(base + c * CHUNK, CHUNK)
        pltpu.sync_copy(feat_hbm.at[pl.ds(tbase, CHUNK)], idx_v)
        pltpu.async_copy(emb_hbm.at[idx_v], rows_v, sem).wait()

        for b in range(B_BAGS):
            lo = jnp.clip(off_s[b] - tbase, 0, CHUNK)
            hi = jnp.clip(off_s[b + 1] - tbase, 0, CHUNK)

            @pl.when(hi > lo)
            def _run():
                def tok_body(t, regs):
                    return tuple(
                        regs[j] + rows_v[t, pl.ds(j * 16, 16)]
                        for j in range(NV)
                    )
                regs = lax.fori_loop(
                    lo, hi, tok_body,
                    tuple(jnp.zeros((16,), jnp.float32) for _ in range(NV)))
                for j in range(NV):
                    sl = pl.ds(j * 16, 16)
                    acc_v[b, sl] = acc_v[b, sl] + regs[j]
        return 0

    lax.fori_loop(0, NCHUNK, chunk_body, 0)
    pltpu.sync_copy(acc_v, out_hbm.at[wid])


def _make_sc_kernel():
    mesh = plsc.VectorSubcoreMesh(core_axis_name="c", subcore_axis_name="s")
    return pl.kernel(
        _sc_body,
        out_type=jax.ShapeDtypeStruct((NW, B_BAGS, EMB), jnp.float32),
        mesh=mesh,
        compiler_params=pltpu.CompilerParams(needs_layout_passes=False),
        scratch_types=[
            pltpu.VMEM((CHUNK,), jnp.int32),
            pltpu.VMEM((CHUNK, EMB), jnp.float32),
            pltpu.VMEM((B_BAGS, EMB), jnp.float32),
            pltpu.VMEM((B_BAGS, 16), jnp.int32),
            pltpu.SemaphoreType.DMA,
        ],
    )


def _tc_body(part_ref, off_ref, lin_ref, out_ref):
    sums = jnp.sum(part_ref[...], axis=0)
    off = off_ref[...]
    nxt = jnp.concatenate(
        [off[:, 1:], jnp.full((1, 1), T_TOKENS, jnp.int32)], axis=1)
    counts = (nxt - off).astype(jnp.float32)
    mean = sums / jnp.maximum(counts, 1.0).reshape(B_BAGS, 1)
    out_ref[...] = lax.dot_general(
        mean, lin_ref[...], (((1,), (1,)), ((), ())),
        preferred_element_type=jnp.float32)


@jax.jit
def kernel(feature_seq, offset_seq, emb_weight, lin_weight):
    zeros = jnp.zeros((B_BAGS, EMB), jnp.float32)
    off_bcast = jnp.broadcast_to(offset_seq[:, None], (B_BAGS, 16))
    partials = _make_sc_kernel()(emb_weight, feature_seq, off_bcast, zeros)
    return pl.pallas_call(
        _tc_body,
        out_shape=jax.ShapeDtypeStruct((B_BAGS, TYPES), jnp.float32),
    )(partials, offset_seq.reshape(1, B_BAGS), lin_weight)


# speedup vs baseline: 7.1208x; 1.8470x over previous
import jax
import jax.numpy as jnp
from jax import lax
from jax.experimental import pallas as pl
from jax.experimental.pallas import tpu as pltpu
from jax.experimental.pallas import tpu_sc as plsc

EMB = 512
NV = EMB // 16
T_TOKENS = 32768
B_BAGS = 16
TYPES = 128
NC = 2
NS = 16
NW = NC * NS
TPW = T_TOKENS // NW
CHUNK = 64
NCHUNK = TPW // CHUNK


NPAIR = NCHUNK // 2


def _sc_body(emb_hbm, feat_hbm, offb_hbm, zeros_hbm, out_hbm,
             idx_all, rows0, rows1, acc_v, offb_v, off_sm, sem0, sem1):
    cid = lax.axis_index("c")
    sid = lax.axis_index("s")
    wid = sid * NC + cid
    base = wid * TPW

    pltpu.sync_copy(offb_hbm, offb_v)
    pltpu.sync_copy(zeros_hbm, acc_v)
    pltpu.sync_copy(feat_hbm.at[pl.ds(base, TPW)], idx_all)

    for b in range(B_BAGS):
        off_sm[b] = lax.reduce_max(offb_v[b, :], (0,))
    off_sm[B_BAGS] = jnp.int32(T_TOKENS)

    def gather(c, rows, sem):
        start = pl.multiple_of(c * CHUNK, CHUNK)
        return pltpu.make_async_copy(
            emb_hbm.at[idx_all.at[pl.ds(start, CHUNK)]], rows, sem)

    def accumulate(rows_v, c):
        tbase = base + c * CHUNK

        def bag_body(b, _):
            lo = jnp.clip(off_sm[b] - tbase, 0, CHUNK)
            hi = jnp.clip(off_sm[b + 1] - tbase, 0, CHUNK)

            @pl.when(hi > lo)
            def _run():
                def tok_body(t, regs):
                    return tuple(
                        regs[j] + rows_v[t, pl.ds(j * 16, 16)]
                        for j in range(NV)
                    )
                regs = lax.fori_loop(
                    lo, hi, tok_body,
                    tuple(jnp.zeros((16,), jnp.float32) for _ in range(NV)))
                for j in range(NV):
                    sl = pl.ds(j * 16, 16)
                    acc_v[b, sl] = acc_v[b, sl] + regs[j]
            return 0

        lax.fori_loop(0, B_BAGS, bag_body, 0)

    gather(0, rows0, sem0).start()

    def pair_body(p, _):
        c0 = 2 * p
        gather(c0 + 1, rows1, sem1).start()
        gather(c0, rows0, sem0).wait()
        accumulate(rows0, c0)

        @pl.when(p < NPAIR - 1)
        def _prefetch():
            gather(c0 + 2, rows0, sem0).start()

        gather(c0 + 1, rows1, sem1).wait()
        accumulate(rows1, c0 + 1)
        return 0

    lax.fori_loop(0, NPAIR, pair_body, 0)
    pltpu.sync_copy(acc_v, out_hbm.at[wid])


def _make_sc_kernel():
    mesh = plsc.VectorSubcoreMesh(core_axis_name="c", subcore_axis_name="s")
    return pl.kernel(
        _sc_body,
        out_type=jax.ShapeDtypeStruct((NW, B_BAGS, EMB), jnp.float32),
        mesh=mesh,
        compiler_params=pltpu.CompilerParams(needs_layout_passes=False),
        scratch_types=[
            pltpu.VMEM((TPW,), jnp.int32),
            pltpu.VMEM((CHUNK, EMB), jnp.float32),
            pltpu.VMEM((CHUNK, EMB), jnp.float32),
            pltpu.VMEM((B_BAGS, EMB), jnp.float32),
            pltpu.VMEM((B_BAGS, 16), jnp.int32),
            pltpu.SMEM((B_BAGS + 1,), jnp.int32),
            pltpu.SemaphoreType.DMA,
            pltpu.SemaphoreType.DMA,
        ],
    )


def _tc_body(part_ref, off_ref, lin_ref, out_ref):
    sums = jnp.sum(part_ref[...], axis=0)
    off = off_ref[...]
    nxt = jnp.concatenate(
        [off[:, 1:], jnp.full((1, 1), T_TOKENS, jnp.int32)], axis=1)
    counts = (nxt - off).astype(jnp.float32)
    mean = sums / jnp.maximum(counts, 1.0).reshape(B_BAGS, 1)
    out_ref[...] = lax.dot_general(
        mean, lin_ref[...], (((1,), (1,)), ((), ())),
        preferred_element_type=jnp.float32)


@jax.jit
def kernel(feature_seq, offset_seq, emb_weight, lin_weight):
    zeros = jnp.zeros((B_BAGS, EMB), jnp.float32)
    off_bcast = jnp.broadcast_to(offset_seq[:, None], (B_BAGS, 16))
    partials = _make_sc_kernel()(emb_weight, feature_seq, off_bcast, zeros)
    return pl.pallas_call(
        _tc_body,
        out_shape=jax.ShapeDtypeStruct((B_BAGS, TYPES), jnp.float32),
    )(partials, offset_seq.reshape(1, B_BAGS), lin_weight)
